# baseline (speedup 1.0000x reference)
import functools

import jax
import jax.numpy as jnp
from jax import lax
from jax.experimental import pallas as pl
from jax.experimental.pallas import tpu as pltpu
from jax.experimental.pallas import tpu_sc as plsc

EMBED_DIM = 64
SCALE = 8.0

_info = plsc.get_sparse_core_info()
NC, NS, L = _info.num_cores, _info.num_subcores, _info.num_lanes
NW = NC * NS

IDX_PER_STREAM = 128
STREAMS_PER_CHUNK = 4
CHUNK = IDX_PER_STREAM * STREAMS_PER_CHUNK


def _emb_body(n_chunks, table_hbm, idx_hbm, out_hbm, idx_v, rows_v, sem):
    wid = lax.axis_index("s") * NC + lax.axis_index("c")

    def chunk_body(g, carry):
        irow = (wid * n_chunks + g) * STREAMS_PER_CHUNK
        row0 = irow * IDX_PER_STREAM
        pltpu.sync_copy(idx_hbm.at[pl.ds(irow, STREAMS_PER_CHUNK)], idx_v)
        copies = [
            pltpu.async_copy(
                table_hbm.at[idx_v.at[j]],
                rows_v.at[pl.ds(j * IDX_PER_STREAM, IDX_PER_STREAM)],
                sem,
            )
            for j in range(STREAMS_PER_CHUNK)
        ]
        for c in copies:
            c.wait()

        def scale_body(r, inner):
            for c in range(EMBED_DIM // L):
                rows_v[r, pl.ds(c * L, L)] = rows_v[r, pl.ds(c * L, L)] * SCALE
            return inner

        lax.fori_loop(0, CHUNK, scale_body, 0, unroll=2)
        pltpu.sync_copy(rows_v, out_hbm.at[pl.ds(row0, CHUNK)])
        return carry

    lax.fori_loop(0, n_chunks, chunk_body, 0)


def kernel(tokens, table):
    n_tok = tokens.shape[0] * tokens.shape[1]
    assert n_tok % (NW * CHUNK) == 0
    n_chunks = n_tok // (NW * CHUNK)
    idx2d = tokens.reshape(n_tok // IDX_PER_STREAM, IDX_PER_STREAM)

    mesh = plsc.VectorSubcoreMesh(core_axis_name="c", subcore_axis_name="s")
    out = pl.kernel(
        functools.partial(_emb_body, n_chunks),
        out_type=jax.ShapeDtypeStruct((n_tok, EMBED_DIM), jnp.float32),
        mesh=mesh,
        scratch_types=[
            pltpu.VMEM((STREAMS_PER_CHUNK, IDX_PER_STREAM), jnp.int32),
            pltpu.VMEM((CHUNK, EMBED_DIM), jnp.float32),
            pltpu.SemaphoreType.DMA,
        ],
        compiler_params=pltpu.CompilerParams(use_tc_tiling_on_sc=False),
    )(table, idx2d)
    return out.reshape(tokens.shape[0], tokens.shape[1], EMBED_DIM)


# speedup vs baseline: 1.0900x; 1.0900x over previous
import functools

import jax
import jax.numpy as jnp
from jax import lax
from jax.experimental import pallas as pl
from jax.experimental.pallas import tpu as pltpu
from jax.experimental.pallas import tpu_sc as plsc

EMBED_DIM = 64
SCALE = 8.0

_info = plsc.get_sparse_core_info()
NC, NS, L = _info.num_cores, _info.num_subcores, _info.num_lanes
NW = NC * NS

IDX_PER_STREAM = 128
STREAMS_PER_CHUNK = 4
CHUNK = IDX_PER_STREAM * STREAMS_PER_CHUNK


def _emb_body(n_chunks, table_hbm, idx_hbm, out_hbm,
              idx_all, buf0, buf1, sem0, sem1):
    wid = lax.axis_index("s") * NC + lax.axis_index("c")
    irows = n_chunks * STREAMS_PER_CHUNK
    irow0 = wid * irows
    row_base = irow0 * IDX_PER_STREAM

    pltpu.sync_copy(idx_hbm.at[pl.ds(irow0, irows)], idx_all)

    def fire(g, buf, sem):
        for j in range(STREAMS_PER_CHUNK):
            pltpu.async_copy(
                table_hbm.at[idx_all.at[g * STREAMS_PER_CHUNK + j]],
                buf.at[pl.ds(j * IDX_PER_STREAM, IDX_PER_STREAM)],
                sem,
            )

    def drain(buf, sem):
        pltpu.make_async_copy(out_hbm.at[pl.ds(0, CHUNK)], buf, sem).wait()

    def scale(buf):
        @plsc.parallel_loop(0, CHUNK, unroll=8)
        def _(r):
            for c in range(EMBED_DIM // L):
                buf[r, pl.ds(c * L, L)] = buf[r, pl.ds(c * L, L)] * SCALE

    def store(g, buf):
        pltpu.sync_copy(buf, out_hbm.at[pl.ds(row_base + g * CHUNK, CHUNK)])

    n_pairs = n_chunks // 2
    fire(0, buf0, sem0)

    def pair(p, carry):
        g0 = p * 2
        fire(g0 + 1, buf1, sem1)
        drain(buf0, sem0)
        scale(buf0)
        store(g0, buf0)
        fire(g0 + 2, buf0, sem0)
        drain(buf1, sem1)
        scale(buf1)
        store(g0 + 1, buf1)
        return carry

    lax.fori_loop(0, n_pairs - 1, pair, 0)

    g0 = (n_pairs - 1) * 2
    fire(g0 + 1, buf1, sem1)
    drain(buf0, sem0)
    scale(buf0)
    store(g0, buf0)
    drain(buf1, sem1)
    scale(buf1)
    store(g0 + 1, buf1)


def kernel(tokens, table):
    n_tok = tokens.shape[0] * tokens.shape[1]
    assert n_tok % (NW * CHUNK * 2) == 0
    n_chunks = n_tok // (NW * CHUNK)
    idx2d = tokens.reshape(n_tok // IDX_PER_STREAM, IDX_PER_STREAM)

    mesh = plsc.VectorSubcoreMesh(core_axis_name="c", subcore_axis_name="s")
    out = pl.kernel(
        functools.partial(_emb_body, n_chunks),
        out_type=jax.ShapeDtypeStruct((n_tok, EMBED_DIM), jnp.float32),
        mesh=mesh,
        scratch_types=[
            pltpu.VMEM((n_chunks * STREAMS_PER_CHUNK, IDX_PER_STREAM),
                       jnp.int32),
            pltpu.VMEM((CHUNK, EMBED_DIM), jnp.float32),
            pltpu.VMEM((CHUNK, EMBED_DIM), jnp.float32),
            pltpu.SemaphoreType.DMA,
            pltpu.SemaphoreType.DMA,
        ],
        compiler_params=pltpu.CompilerParams(use_tc_tiling_on_sc=False),
    )(table, idx2d)
    return out.reshape(tokens.shape[0], tokens.shape[1], EMBED_DIM)


# speedup vs baseline: 1.3289x; 1.2193x over previous
import functools

import jax
import jax.numpy as jnp
from jax import lax
from jax.experimental import pallas as pl
from jax.experimental.pallas import tpu as pltpu
from jax.experimental.pallas import tpu_sc as plsc

EMBED_DIM = 64
PAD_DIM = 128
SCALE = 8.0

_info = plsc.get_sparse_core_info()
NC, NS, L = _info.num_cores, _info.num_subcores, _info.num_lanes
NW = NC * NS

IDX_PER_STREAM = 128
STREAMS_PER_CHUNK = 2
CHUNK = IDX_PER_STREAM * STREAMS_PER_CHUNK


def _emb_body(n_chunks, table_hbm, idx_hbm, out_hbm,
              idx_all, buf0, buf1, sem0, sem1):
    wid = lax.axis_index("s") * NC + lax.axis_index("c")
    irows = n_chunks * STREAMS_PER_CHUNK
    irow0 = wid * irows
    row_base = irow0 * IDX_PER_STREAM

    pltpu.sync_copy(idx_hbm.at[pl.ds(irow0, irows)], idx_all)

    def fire(g, buf, sem):
        for j in range(STREAMS_PER_CHUNK):
            pltpu.async_copy(
                table_hbm.at[idx_all.at[g * STREAMS_PER_CHUNK + j]],
                buf.at[pl.ds(j * IDX_PER_STREAM, IDX_PER_STREAM)],
                sem,
            )

    def drain(buf, sem):
        pltpu.make_async_copy(table_hbm.at[pl.ds(0, CHUNK)], buf, sem).wait()

    def scale(buf):
        @plsc.parallel_loop(0, CHUNK, unroll=8)
        def _(r):
            for c in range(EMBED_DIM // L):
                buf[r, pl.ds(c * L, L)] = buf[r, pl.ds(c * L, L)] * SCALE

    def store(g, buf):
        pltpu.sync_copy(buf, out_hbm.at[pl.ds(row_base + g * CHUNK, CHUNK)])

    n_pairs = n_chunks // 2
    fire(0, buf0, sem0)

    def pair(p, carry):
        g0 = p * 2
        fire(g0 + 1, buf1, sem1)
        drain(buf0, sem0)
        scale(buf0)
        store(g0, buf0)
        fire(g0 + 2, buf0, sem0)
        drain(buf1, sem1)
        scale(buf1)
        store(g0 + 1, buf1)
        return carry

    lax.fori_loop(0, n_pairs - 1, pair, 0)

    g0 = (n_pairs - 1) * 2
    fire(g0 + 1, buf1, sem1)
    drain(buf0, sem0)
    scale(buf0)
    store(g0, buf0)
    drain(buf1, sem1)
    scale(buf1)
    store(g0 + 1, buf1)


def kernel(tokens, table):
    n_tok = tokens.shape[0] * tokens.shape[1]
    assert n_tok % (NW * CHUNK * 2) == 0
    n_chunks = n_tok // (NW * CHUNK)
    idx2d = tokens.reshape(n_tok // IDX_PER_STREAM, IDX_PER_STREAM)
    vocab = table.shape[0]
    table128 = jnp.concatenate(
        [table, jnp.zeros((vocab, PAD_DIM - EMBED_DIM), table.dtype)], axis=1)

    mesh = plsc.VectorSubcoreMesh(core_axis_name="c", subcore_axis_name="s")
    out = pl.kernel(
        functools.partial(_emb_body, n_chunks),
        out_type=jax.ShapeDtypeStruct((n_tok, PAD_DIM), jnp.float32),
        mesh=mesh,
        scratch_types=[
            pltpu.VMEM((n_chunks * STREAMS_PER_CHUNK, IDX_PER_STREAM),
                       jnp.int32),
            pltpu.VMEM((CHUNK, PAD_DIM), jnp.float32),
            pltpu.VMEM((CHUNK, PAD_DIM), jnp.float32),
            pltpu.SemaphoreType.DMA,
            pltpu.SemaphoreType.DMA,
        ],
        compiler_params=pltpu.CompilerParams(use_tc_tiling_on_sc=True),
    )(table128, idx2d)
    out = out[:, :EMBED_DIM]
    return out.reshape(tokens.shape[0], tokens.shape[1], EMBED_DIM)
